# baseline (speedup 1.0000x reference)
import functools

import jax
import jax.numpy as jnp
from jax import lax
from jax.experimental import pallas as pl
from jax.experimental.pallas import tpu as pltpu
from jax.experimental.pallas import tpu_sc as plsc

N = 10000
NP = 10240
E = 320000
EP = 327680
NC = 2
NS = 16
NW = NC * NS
EPT = EP // NW
B = 128
NB = EPT // B
RPT = NP // NS
D_IN = 128
D_HID = 256
GB = 1024
NG = NP // GB

_mesh = plsc.VectorSubcoreMesh(
    core_axis_name="c", subcore_axis_name="s", num_cores=NC, num_subcores=NS
)


def _deg_body(dst_hbm, out_hbm, dst_v, buf_v, acc_sh):
    c = lax.axis_index("c")
    s = lax.axis_index("s")
    wid = s * NC + c

    fz = jnp.zeros((16,), jnp.float32)
    fo = jnp.ones((16,), jnp.float32)

    def zloop(i, _):
        buf_v[i // 8, pl.ds((i % 8) * 16, 16)] = fz
        return 0

    lax.fori_loop(0, B * D_IN // 16, zloop, 0)

    for j in range(RPT // B):
        pltpu.sync_copy(buf_v, acc_sh.at[pl.ds(s * RPT + j * B, B)])

    def oloop(i, _):
        buf_v[i // 8, pl.ds((i % 8) * 16, 16)] = fo
        return 0

    lax.fori_loop(0, B * D_IN // 16, oloop, 0)

    pltpu.sync_copy(dst_hbm.at[wid], dst_v)
    plsc.subcore_barrier()

    def dloop(b, _):
        pltpu.sync_copy(buf_v, acc_sh.at[dst_v.at[b]], add=True)
        return 0

    lax.fori_loop(0, NB, dloop, 0)
    plsc.subcore_barrier()

    pltpu.sync_copy(acc_sh.at[pl.ds(s * RPT, RPT)], out_hbm.at[c, pl.ds(s * RPT, RPT)])


_deg_call = pl.kernel(
    _deg_body,
    out_type=jax.ShapeDtypeStruct((NC, NP, D_IN), jnp.float32),
    mesh=_mesh,
    scratch_types=[
        pltpu.VMEM((NB, B), jnp.int32),
        pltpu.VMEM((B, D_IN), jnp.float32),
        pltpu.VMEM_SHARED((NP, D_IN), jnp.float32),
    ],
)


def _seg_body(table_hbm, src_hbm, dst_hbm, out_hbm, src_v, dst_v, buf_v, acc_sh):
    c = lax.axis_index("c")
    s = lax.axis_index("s")
    wid = s * NC + c

    fz = jnp.zeros((16,), jnp.float32)

    def zloop(i, _):
        buf_v[i // 8, pl.ds((i % 8) * 16, 16)] = fz
        return 0

    lax.fori_loop(0, B * D_IN // 16, zloop, 0)

    for j in range(RPT // B):
        pltpu.sync_copy(buf_v, acc_sh.at[pl.ds(s * RPT + j * B, B)])

    pltpu.sync_copy(src_hbm.at[wid], src_v)
    pltpu.sync_copy(dst_hbm.at[wid], dst_v)
    plsc.subcore_barrier()

    def body(b, _):
        pltpu.sync_copy(table_hbm.at[src_v.at[b]], buf_v)
        pltpu.sync_copy(buf_v, acc_sh.at[dst_v.at[b]], add=True)
        return 0

    lax.fori_loop(0, NB, body, 0)
    plsc.subcore_barrier()

    pltpu.sync_copy(acc_sh.at[pl.ds(s * RPT, RPT)], out_hbm.at[c, pl.ds(s * RPT, RPT)])


_seg_call = pl.kernel(
    _seg_body,
    out_type=jax.ShapeDtypeStruct((NC, NP, D_IN), jnp.float32),
    mesh=_mesh,
    scratch_types=[
        pltpu.VMEM((NB, B), jnp.int32),
        pltpu.VMEM((NB, B), jnp.int32),
        pltpu.VMEM((B, D_IN), jnp.float32),
        pltpu.VMEM_SHARED((NP, D_IN), jnp.float32),
    ],
)


def _prep_body(dp_ref, x_ref, w1_ref, dinv_ref, y0_ref, y1_ref):
    deg = dp_ref[0][:, :1] + dp_ref[1][:, :1] + 1.0
    dinv = 1.0 / jnp.sqrt(deg)
    dinv_ref[...] = dinv
    xw = jnp.dot(x_ref[...], w1_ref[...], preferred_element_type=jnp.float32)
    y = xw * dinv
    y0_ref[...] = y[:, :D_IN]
    y1_ref[...] = y[:, D_IN:]


_prep_call = pl.pallas_call(
    _prep_body,
    grid=(NG,),
    in_specs=[
        pl.BlockSpec((NC, GB, D_IN), lambda i: (0, i, 0)),
        pl.BlockSpec((GB, D_IN), lambda i: (i, 0)),
        pl.BlockSpec((D_IN, D_HID), lambda i: (0, 0)),
    ],
    out_specs=[
        pl.BlockSpec((GB, 1), lambda i: (i, 0)),
        pl.BlockSpec((GB, D_IN), lambda i: (i, 0)),
        pl.BlockSpec((GB, D_IN), lambda i: (i, 0)),
    ],
    out_shape=[
        jax.ShapeDtypeStruct((NP, 1), jnp.float32),
        jax.ShapeDtypeStruct((NP, D_IN), jnp.float32),
        jax.ShapeDtypeStruct((NP, D_IN), jnp.float32),
    ],
)


def _mid_body(a0_ref, a1_ref, y0_ref, y1_ref, dinv_ref, b1_ref, w2_ref,
              t0_ref, t1_ref):
    a0 = a0_ref[0] + a0_ref[1] + y0_ref[...]
    a1 = a1_ref[0] + a1_ref[1] + y1_ref[...]
    agg = jnp.concatenate([a0, a1], axis=1)
    h1 = jnp.maximum(agg * dinv_ref[...] + b1_ref[...], 0.0)
    xw = jnp.dot(h1, w2_ref[...], preferred_element_type=jnp.float32)
    t = xw * dinv_ref[...]
    t0_ref[...] = t[:, :D_IN]
    t1_ref[...] = t[:, D_IN:]


_mid_call = pl.pallas_call(
    _mid_body,
    grid=(NG,),
    in_specs=[
        pl.BlockSpec((NC, GB, D_IN), lambda i: (0, i, 0)),
        pl.BlockSpec((NC, GB, D_IN), lambda i: (0, i, 0)),
        pl.BlockSpec((GB, D_IN), lambda i: (i, 0)),
        pl.BlockSpec((GB, D_IN), lambda i: (i, 0)),
        pl.BlockSpec((GB, 1), lambda i: (i, 0)),
        pl.BlockSpec((1, D_HID), lambda i: (0, 0)),
        pl.BlockSpec((D_HID, D_HID), lambda i: (0, 0)),
    ],
    out_specs=[
        pl.BlockSpec((GB, D_IN), lambda i: (i, 0)),
        pl.BlockSpec((GB, D_IN), lambda i: (i, 0)),
    ],
    out_shape=[
        jax.ShapeDtypeStruct((NP, D_IN), jnp.float32),
        jax.ShapeDtypeStruct((NP, D_IN), jnp.float32),
    ],
)


def _head_body(a0_ref, a1_ref, y0_ref, y1_ref, dinv_ref, b2_ref,
               wf1_ref, bf1_ref, wf2_ref, bf2_ref, o_ref):
    a0 = a0_ref[0] + a0_ref[1] + y0_ref[...]
    a1 = a1_ref[0] + a1_ref[1] + y1_ref[...]
    agg = jnp.concatenate([a0, a1], axis=1)
    h2 = jnp.maximum(agg * dinv_ref[...] + b2_ref[...], 0.0)
    h3 = jnp.dot(h2, wf1_ref[...], preferred_element_type=jnp.float32) + bf1_ref[...]
    h3 = jnp.maximum(h3, 0.0)
    o_ref[...] = jnp.dot(h3, wf2_ref[...], preferred_element_type=jnp.float32) + bf2_ref[...]


_head_call = pl.pallas_call(
    _head_body,
    grid=(NG,),
    in_specs=[
        pl.BlockSpec((NC, GB, D_IN), lambda i: (0, i, 0)),
        pl.BlockSpec((NC, GB, D_IN), lambda i: (0, i, 0)),
        pl.BlockSpec((GB, D_IN), lambda i: (i, 0)),
        pl.BlockSpec((GB, D_IN), lambda i: (i, 0)),
        pl.BlockSpec((GB, 1), lambda i: (i, 0)),
        pl.BlockSpec((1, D_HID), lambda i: (0, 0)),
        pl.BlockSpec((D_HID, D_HID // 2), lambda i: (0, 0)),
        pl.BlockSpec((1, D_HID // 2), lambda i: (0, 0)),
        pl.BlockSpec((D_HID // 2, 1), lambda i: (0, 0)),
        pl.BlockSpec((1, 1), lambda i: (0, 0)),
    ],
    out_specs=pl.BlockSpec((GB, 1), lambda i: (i, 0)),
    out_shape=jax.ShapeDtypeStruct((NP, 1), jnp.float32),
)


def kernel(x, edge_index, W1, b1, W2, b2, Wf1, bf1, Wf2, bf2):
    ei = edge_index.astype(jnp.int32)
    pad = N + (jnp.arange(EP - E, dtype=jnp.int32) % (NP - N))
    srcp = jnp.concatenate([ei[0], pad]).reshape(NW, NB, B)
    dstp = jnp.concatenate([ei[1], pad]).reshape(NW, NB, B)
    x_pad = jnp.zeros((NP, D_IN), jnp.float32).at[:N].set(x)

    degp = _deg_call(dstp)
    dinv, y10, y11 = _prep_call(degp, x_pad, W1)
    a10p = _seg_call(y10, srcp, dstp)
    a11p = _seg_call(y11, srcp, dstp)
    y20, y21 = _mid_call(a10p, a11p, y10, y11, dinv, b1.reshape(1, -1), W2)
    a20p = _seg_call(y20, srcp, dstp)
    a21p = _seg_call(y21, srcp, dstp)
    o = _head_call(a20p, a21p, y20, y21, dinv, b2.reshape(1, -1),
                   Wf1, bf1.reshape(1, -1), Wf2, bf2.reshape(1, -1))
    return o[:N, 0]


# speedup vs baseline: 1.2755x; 1.2755x over previous
import functools

import jax
import jax.numpy as jnp
from jax import lax
from jax.experimental import pallas as pl
from jax.experimental.pallas import tpu as pltpu
from jax.experimental.pallas import tpu_sc as plsc

N = 10000
NP = 10240
E = 320000
EP = 327680
NC = 2
NS = 16
NW = NC * NS
EPT = EP // NW
B = 128
NB = EPT // B
RPT = NP // NS
NBUF = 4
D_IN = 128
D_HID = 256
GB = 1024
NG = NP // GB

_mesh = plsc.VectorSubcoreMesh(
    core_axis_name="c", subcore_axis_name="s", num_cores=NC, num_subcores=NS
)


def _deg_body(dst_hbm, out_hbm, dst_v, buf_v, acc_sh, dsem):
    c = lax.axis_index("c")
    s = lax.axis_index("s")
    wid = s * NC + c

    fz = jnp.zeros((16,), jnp.float32)
    fo = jnp.ones((16,), jnp.float32)

    def zloop(i, _):
        buf_v[i // 8, pl.ds((i % 8) * 16, 16)] = fz
        return 0

    lax.fori_loop(0, B * D_IN // 16, zloop, 0)

    for j in range(RPT // B):
        pltpu.sync_copy(buf_v, acc_sh.at[pl.ds(s * RPT + j * B, B)])

    def oloop(i, _):
        buf_v[i // 8, pl.ds((i % 8) * 16, 16)] = fo
        return 0

    lax.fori_loop(0, B * D_IN // 16, oloop, 0)

    pltpu.sync_copy(dst_hbm.at[wid], dst_v)
    plsc.subcore_barrier()

    for j in range(NBUF):
        pltpu.async_copy(buf_v, acc_sh.at[dst_v.at[j]], dsem, add=True)

    def dloop(b, _):
        pltpu.make_async_copy(buf_v, acc_sh.at[dst_v.at[b]], dsem).wait()
        pltpu.async_copy(buf_v, acc_sh.at[dst_v.at[b + NBUF]], dsem, add=True)
        return 0

    lax.fori_loop(0, NB - NBUF, dloop, 0)
    for j in range(NBUF):
        pltpu.make_async_copy(buf_v, acc_sh.at[dst_v.at[j]], dsem).wait()
    plsc.subcore_barrier()

    pltpu.sync_copy(acc_sh.at[pl.ds(s * RPT, RPT)], out_hbm.at[c, pl.ds(s * RPT, RPT)])


_deg_call = pl.kernel(
    _deg_body,
    out_type=jax.ShapeDtypeStruct((NC, NP, D_IN), jnp.float32),
    mesh=_mesh,
    scratch_types=[
        pltpu.VMEM((NB, B), jnp.int32),
        pltpu.VMEM((B, D_IN), jnp.float32),
        pltpu.VMEM_SHARED((NP, D_IN), jnp.float32),
        pltpu.SemaphoreType.DMA,
    ],
)


CHK = 16
CHN = NB // CHK


def _seg_body(table_hbm, src_hbm, dst_hbm, out_hbm, src_v, dst_v,
              buf0, buf1, acc_sh, g0, g1, s0, s1):
    c = lax.axis_index("c")
    s = lax.axis_index("s")
    wid = s * NC + c

    fz = jnp.zeros((16,), jnp.float32)

    def g_issue(b, buf, sem):
        pltpu.async_copy(table_hbm.at[src_v.at[b]], buf, sem)

    def g_wait(b, buf, sem):
        pltpu.make_async_copy(table_hbm.at[src_v.at[b]], buf, sem).wait()

    def s_issue(q, kk, buf, sem):
        pltpu.async_copy(buf, acc_sh.at[dst_v.at[q, kk]], sem, add=True)

    def s_wait(q, kk, buf, sem):
        pltpu.make_async_copy(buf, acc_sh.at[dst_v.at[q, kk]], sem).wait()

    def zloop(i, _):
        buf0[i // 8, pl.ds((i % 8) * 16, 16)] = fz
        return 0

    lax.fori_loop(0, B * D_IN // 16, zloop, 0)

    for j in range(RPT // B):
        pltpu.sync_copy(buf0, acc_sh.at[pl.ds(s * RPT + j * B, B)])

    pltpu.sync_copy(src_hbm.at[wid], src_v)
    pltpu.sync_copy(dst_hbm.at[wid, pl.ds(0, CHK)], dst_v.at[0])
    g_issue(0, buf0, g0)
    plsc.subcore_barrier()

    def outer(ch, _):
        q = ch % 2

        @pl.when(ch > 0)
        def _():
            pltpu.sync_copy(dst_hbm.at[wid, pl.ds(ch * CHK, CHK)], dst_v.at[q])

        def inner(k, _):
            b0 = ch * CHK + 2 * k
            b1 = b0 + 1
            g_wait(b0, buf0, g0)
            s_issue(q, 2 * k, buf0, s0)

            @pl.when(b0 > 0)
            def _():
                s_wait(q, 2 * k, buf1, s1)

            g_issue(b1, buf1, g1)
            g_wait(b1, buf1, g1)
            s_issue(q, 2 * k + 1, buf1, s1)
            s_wait(q, 2 * k, buf0, s0)

            @pl.when(b1 + 1 < NB)
            def _():
                g_issue(b1 + 1, buf0, g0)

            return 0

        lax.fori_loop(0, CHK // 2, inner, 0)
        return 0

    lax.fori_loop(0, CHN, outer, 0)
    s_wait(0, 0, buf1, s1)
    plsc.subcore_barrier()

    pltpu.sync_copy(acc_sh.at[pl.ds(s * RPT, RPT)], out_hbm.at[c, pl.ds(s * RPT, RPT)])


_seg_call = pl.kernel(
    _seg_body,
    out_type=jax.ShapeDtypeStruct((NC, NP, D_IN), jnp.float32),
    mesh=_mesh,
    scratch_types=[
        pltpu.VMEM((NB, B), jnp.int32),
        pltpu.VMEM((2, CHK, B), jnp.int32),
        pltpu.VMEM((B, D_IN), jnp.float32),
        pltpu.VMEM((B, D_IN), jnp.float32),
        pltpu.VMEM_SHARED((NP, D_IN), jnp.float32),
        pltpu.SemaphoreType.DMA,
        pltpu.SemaphoreType.DMA,
        pltpu.SemaphoreType.DMA,
        pltpu.SemaphoreType.DMA,
    ],
)


def _prep_body(dp_ref, x_ref, w1_ref, dinv_ref, y0_ref, y1_ref):
    deg = dp_ref[0][:, :1] + dp_ref[1][:, :1] + 1.0
    dinv = 1.0 / jnp.sqrt(deg)
    dinv_ref[...] = dinv
    xw = jnp.dot(x_ref[...], w1_ref[...], preferred_element_type=jnp.float32)
    y = xw * dinv
    y0_ref[...] = y[:, :D_IN]
    y1_ref[...] = y[:, D_IN:]


_prep_call = pl.pallas_call(
    _prep_body,
    grid=(NG,),
    in_specs=[
        pl.BlockSpec((NC, GB, D_IN), lambda i: (0, i, 0)),
        pl.BlockSpec((GB, D_IN), lambda i: (i, 0)),
        pl.BlockSpec((D_IN, D_HID), lambda i: (0, 0)),
    ],
    out_specs=[
        pl.BlockSpec((GB, 1), lambda i: (i, 0)),
        pl.BlockSpec((GB, D_IN), lambda i: (i, 0)),
        pl.BlockSpec((GB, D_IN), lambda i: (i, 0)),
    ],
    out_shape=[
        jax.ShapeDtypeStruct((NP, 1), jnp.float32),
        jax.ShapeDtypeStruct((NP, D_IN), jnp.float32),
        jax.ShapeDtypeStruct((NP, D_IN), jnp.float32),
    ],
)


def _mid_body(a0_ref, a1_ref, y0_ref, y1_ref, dinv_ref, b1_ref, w2_ref,
              t0_ref, t1_ref):
    a0 = a0_ref[0] + a0_ref[1] + y0_ref[...]
    a1 = a1_ref[0] + a1_ref[1] + y1_ref[...]
    agg = jnp.concatenate([a0, a1], axis=1)
    h1 = jnp.maximum(agg * dinv_ref[...] + b1_ref[...], 0.0)
    xw = jnp.dot(h1, w2_ref[...], preferred_element_type=jnp.float32)
    t = xw * dinv_ref[...]
    t0_ref[...] = t[:, :D_IN]
    t1_ref[...] = t[:, D_IN:]


_mid_call = pl.pallas_call(
    _mid_body,
    grid=(NG,),
    in_specs=[
        pl.BlockSpec((NC, GB, D_IN), lambda i: (0, i, 0)),
        pl.BlockSpec((NC, GB, D_IN), lambda i: (0, i, 0)),
        pl.BlockSpec((GB, D_IN), lambda i: (i, 0)),
        pl.BlockSpec((GB, D_IN), lambda i: (i, 0)),
        pl.BlockSpec((GB, 1), lambda i: (i, 0)),
        pl.BlockSpec((1, D_HID), lambda i: (0, 0)),
        pl.BlockSpec((D_HID, D_HID), lambda i: (0, 0)),
    ],
    out_specs=[
        pl.BlockSpec((GB, D_IN), lambda i: (i, 0)),
        pl.BlockSpec((GB, D_IN), lambda i: (i, 0)),
    ],
    out_shape=[
        jax.ShapeDtypeStruct((NP, D_IN), jnp.float32),
        jax.ShapeDtypeStruct((NP, D_IN), jnp.float32),
    ],
)


def _head_body(a0_ref, a1_ref, y0_ref, y1_ref, dinv_ref, b2_ref,
               wf1_ref, bf1_ref, wf2_ref, bf2_ref, o_ref):
    a0 = a0_ref[0] + a0_ref[1] + y0_ref[...]
    a1 = a1_ref[0] + a1_ref[1] + y1_ref[...]
    agg = jnp.concatenate([a0, a1], axis=1)
    h2 = jnp.maximum(agg * dinv_ref[...] + b2_ref[...], 0.0)
    h3 = jnp.dot(h2, wf1_ref[...], preferred_element_type=jnp.float32) + bf1_ref[...]
    h3 = jnp.maximum(h3, 0.0)
    o_ref[...] = jnp.dot(h3, wf2_ref[...], preferred_element_type=jnp.float32) + bf2_ref[...]


_head_call = pl.pallas_call(
    _head_body,
    grid=(NG,),
    in_specs=[
        pl.BlockSpec((NC, GB, D_IN), lambda i: (0, i, 0)),
        pl.BlockSpec((NC, GB, D_IN), lambda i: (0, i, 0)),
        pl.BlockSpec((GB, D_IN), lambda i: (i, 0)),
        pl.BlockSpec((GB, D_IN), lambda i: (i, 0)),
        pl.BlockSpec((GB, 1), lambda i: (i, 0)),
        pl.BlockSpec((1, D_HID), lambda i: (0, 0)),
        pl.BlockSpec((D_HID, D_HID // 2), lambda i: (0, 0)),
        pl.BlockSpec((1, D_HID // 2), lambda i: (0, 0)),
        pl.BlockSpec((D_HID // 2, 1), lambda i: (0, 0)),
        pl.BlockSpec((1, 1), lambda i: (0, 0)),
    ],
    out_specs=pl.BlockSpec((GB, 1), lambda i: (i, 0)),
    out_shape=jax.ShapeDtypeStruct((NP, 1), jnp.float32),
)


def kernel(x, edge_index, W1, b1, W2, b2, Wf1, bf1, Wf2, bf2):
    ei = edge_index.astype(jnp.int32)
    pad = N + (jnp.arange(EP - E, dtype=jnp.int32) % (NP - N))
    srcp = jnp.concatenate([ei[0], pad]).reshape(NW, NB, B)
    dstp = jnp.concatenate([ei[1], pad]).reshape(NW, NB, B)
    x_pad = jnp.zeros((NP, D_IN), jnp.float32).at[:N].set(x)

    degp = _deg_call(dstp)
    dinv, y10, y11 = _prep_call(degp, x_pad, W1)
    a10p = _seg_call(y10, srcp, dstp)
    a11p = _seg_call(y11, srcp, dstp)
    y20, y21 = _mid_call(a10p, a11p, y10, y11, dinv, b1.reshape(1, -1), W2)
    a20p = _seg_call(y20, srcp, dstp)
    a21p = _seg_call(y21, srcp, dstp)
    o = _head_call(a20p, a21p, y20, y21, dinv, b2.reshape(1, -1),
                   Wf1, bf1.reshape(1, -1), Wf2, bf2.reshape(1, -1))
    return o[:N, 0]


# speedup vs baseline: 1.2935x; 1.0141x over previous
import functools

import jax
import jax.numpy as jnp
from jax import lax
from jax.experimental import pallas as pl
from jax.experimental.pallas import tpu as pltpu
from jax.experimental.pallas import tpu_sc as plsc

N = 10000
NP = 10240
E = 320000
EP = 327680
NC = 2
NS = 16
NW = NC * NS
EPT = EP // NW
B = 128
NB = EPT // B
RPT = NP // NS
NBUF = 4
D_IN = 128
D_HID = 256
GB = 1024
NG = NP // GB

_mesh = plsc.VectorSubcoreMesh(
    core_axis_name="c", subcore_axis_name="s", num_cores=NC, num_subcores=NS
)


def _deg_body(dst_hbm, out_hbm, dst_v, buf_v, acc_sh, dsem):
    c = lax.axis_index("c")
    s = lax.axis_index("s")
    wid = s * NC + c

    fz = jnp.zeros((16,), jnp.float32)
    fo = jnp.ones((16,), jnp.float32)

    def zloop(i, _):
        buf_v[i // 8, pl.ds((i % 8) * 16, 16)] = fz
        return 0

    lax.fori_loop(0, B * D_IN // 16, zloop, 0)

    for j in range(RPT // B):
        pltpu.sync_copy(buf_v, acc_sh.at[pl.ds(s * RPT + j * B, B)])

    def oloop(i, _):
        buf_v[i // 8, pl.ds((i % 8) * 16, 16)] = fo
        return 0

    lax.fori_loop(0, B * D_IN // 16, oloop, 0)

    pltpu.sync_copy(dst_hbm.at[wid], dst_v)
    plsc.subcore_barrier()

    for j in range(NBUF):
        pltpu.async_copy(buf_v, acc_sh.at[dst_v.at[j]], dsem, add=True)

    def dloop(b, _):
        pltpu.make_async_copy(buf_v, acc_sh.at[dst_v.at[b]], dsem).wait()
        pltpu.async_copy(buf_v, acc_sh.at[dst_v.at[b + NBUF]], dsem, add=True)
        return 0

    lax.fori_loop(0, NB - NBUF, dloop, 0)
    for j in range(NBUF):
        pltpu.make_async_copy(buf_v, acc_sh.at[dst_v.at[j]], dsem).wait()
    plsc.subcore_barrier()

    pltpu.sync_copy(acc_sh.at[pl.ds(s * RPT, RPT)], out_hbm.at[c, pl.ds(s * RPT, RPT)])


_deg_call = pl.kernel(
    _deg_body,
    out_type=jax.ShapeDtypeStruct((NC, NP, D_IN), jnp.float32),
    mesh=_mesh,
    scratch_types=[
        pltpu.VMEM((NB, B), jnp.int32),
        pltpu.VMEM((B, D_IN), jnp.float32),
        pltpu.VMEM_SHARED((NP, D_IN), jnp.float32),
        pltpu.SemaphoreType.DMA,
    ],
)


CHK = 16
CHN = NB // CHK


def _seg_body(t0_hbm, t1_hbm, src_hbm, dst_hbm, o0_hbm, o1_hbm, src_v, dst_v,
              buf0, buf1, acc_sh, g0, g1, s0, s1):
    c = lax.axis_index("c")
    s = lax.axis_index("s")
    wid = s * NC + c

    fz = jnp.zeros((16,), jnp.float32)

    pltpu.sync_copy(src_hbm.at[wid], src_v)

    def do_pass(table_hbm, out_hbm):
        def g_issue(b, buf, sem):
            pltpu.async_copy(table_hbm.at[src_v.at[b]], buf, sem)

        def g_wait(b, buf, sem):
            pltpu.make_async_copy(table_hbm.at[src_v.at[b]], buf, sem).wait()

        def s_issue(q, kk, buf, sem):
            pltpu.async_copy(buf, acc_sh.at[dst_v.at[q, kk]], sem, add=True)

        def s_wait(q, kk, buf, sem):
            pltpu.make_async_copy(buf, acc_sh.at[dst_v.at[q, kk]], sem).wait()

        def zloop(i, _):
            buf0[i // 8, pl.ds((i % 8) * 16, 16)] = fz
            return 0

        lax.fori_loop(0, B * D_IN // 16, zloop, 0)

        for j in range(RPT // B):
            pltpu.sync_copy(buf0, acc_sh.at[pl.ds(s * RPT + j * B, B)])

        g_issue(0, buf0, g0)
        plsc.subcore_barrier()

        def outer(ch, _):
            q = ch % 2
            pltpu.sync_copy(dst_hbm.at[wid, pl.ds(ch * CHK, CHK)], dst_v.at[q])

            def inner(k, _):
                b0 = ch * CHK + 2 * k
                b1 = b0 + 1
                g_wait(b0, buf0, g0)
                s_issue(q, 2 * k, buf0, s0)

                @pl.when(b0 > 0)
                def _():
                    s_wait(q, 2 * k, buf1, s1)

                g_issue(b1, buf1, g1)
                g_wait(b1, buf1, g1)
                s_issue(q, 2 * k + 1, buf1, s1)
                s_wait(q, 2 * k, buf0, s0)

                @pl.when(b1 + 1 < NB)
                def _():
                    g_issue(b1 + 1, buf0, g0)

                return 0

            lax.fori_loop(0, CHK // 2, inner, 0)
            return 0

        lax.fori_loop(0, CHN, outer, 0)
        s_wait(0, 0, buf1, s1)
        plsc.subcore_barrier()

        pltpu.sync_copy(acc_sh.at[pl.ds(s * RPT, RPT)],
                        out_hbm.at[c, pl.ds(s * RPT, RPT)])

    do_pass(t0_hbm, o0_hbm)
    do_pass(t1_hbm, o1_hbm)


_seg_call = pl.kernel(
    _seg_body,
    out_type=(
        jax.ShapeDtypeStruct((NC, NP, D_IN), jnp.float32),
        jax.ShapeDtypeStruct((NC, NP, D_IN), jnp.float32),
    ),
    mesh=_mesh,
    scratch_types=[
        pltpu.VMEM((NB, B), jnp.int32),
        pltpu.VMEM((2, CHK, B), jnp.int32),
        pltpu.VMEM((B, D_IN), jnp.float32),
        pltpu.VMEM((B, D_IN), jnp.float32),
        pltpu.VMEM_SHARED((NP, D_IN), jnp.float32),
        pltpu.SemaphoreType.DMA,
        pltpu.SemaphoreType.DMA,
        pltpu.SemaphoreType.DMA,
        pltpu.SemaphoreType.DMA,
    ],
)


def _mm1_body(x_ref, w1_ref, xw_ref):
    xw_ref[...] = jnp.dot(x_ref[...], w1_ref[...],
                          preferred_element_type=jnp.float32)


_mm1_call = pl.pallas_call(
    _mm1_body,
    grid=(NG,),
    in_specs=[
        pl.BlockSpec((GB, D_IN), lambda i: (i, 0)),
        pl.BlockSpec((D_IN, D_HID), lambda i: (0, 0)),
    ],
    out_specs=pl.BlockSpec((GB, D_HID), lambda i: (i, 0)),
    out_shape=jax.ShapeDtypeStruct((NP, D_HID), jnp.float32),
)


def _prep_body(dp_ref, xw_ref, dinv_ref, y0_ref, y1_ref):
    deg = dp_ref[0][:, :1] + dp_ref[1][:, :1] + 1.0
    dinv = 1.0 / jnp.sqrt(deg)
    dinv_ref[...] = dinv
    y = xw_ref[...] * dinv
    y0_ref[...] = y[:, :D_IN]
    y1_ref[...] = y[:, D_IN:]


_prep_call = pl.pallas_call(
    _prep_body,
    grid=(NG,),
    in_specs=[
        pl.BlockSpec((NC, GB, D_IN), lambda i: (0, i, 0)),
        pl.BlockSpec((GB, D_HID), lambda i: (i, 0)),
    ],
    out_specs=[
        pl.BlockSpec((GB, 1), lambda i: (i, 0)),
        pl.BlockSpec((GB, D_IN), lambda i: (i, 0)),
        pl.BlockSpec((GB, D_IN), lambda i: (i, 0)),
    ],
    out_shape=[
        jax.ShapeDtypeStruct((NP, 1), jnp.float32),
        jax.ShapeDtypeStruct((NP, D_IN), jnp.float32),
        jax.ShapeDtypeStruct((NP, D_IN), jnp.float32),
    ],
)


def _mid_body(a0_ref, a1_ref, y0_ref, y1_ref, dinv_ref, b1_ref, w2_ref,
              t0_ref, t1_ref):
    a0 = a0_ref[0] + a0_ref[1] + y0_ref[...]
    a1 = a1_ref[0] + a1_ref[1] + y1_ref[...]
    agg = jnp.concatenate([a0, a1], axis=1)
    h1 = jnp.maximum(agg * dinv_ref[...] + b1_ref[...], 0.0)
    xw = jnp.dot(h1, w2_ref[...], preferred_element_type=jnp.float32)
    t = xw * dinv_ref[...]
    t0_ref[...] = t[:, :D_IN]
    t1_ref[...] = t[:, D_IN:]


_mid_call = pl.pallas_call(
    _mid_body,
    grid=(NG,),
    in_specs=[
        pl.BlockSpec((NC, GB, D_IN), lambda i: (0, i, 0)),
        pl.BlockSpec((NC, GB, D_IN), lambda i: (0, i, 0)),
        pl.BlockSpec((GB, D_IN), lambda i: (i, 0)),
        pl.BlockSpec((GB, D_IN), lambda i: (i, 0)),
        pl.BlockSpec((GB, 1), lambda i: (i, 0)),
        pl.BlockSpec((1, D_HID), lambda i: (0, 0)),
        pl.BlockSpec((D_HID, D_HID), lambda i: (0, 0)),
    ],
    out_specs=[
        pl.BlockSpec((GB, D_IN), lambda i: (i, 0)),
        pl.BlockSpec((GB, D_IN), lambda i: (i, 0)),
    ],
    out_shape=[
        jax.ShapeDtypeStruct((NP, D_IN), jnp.float32),
        jax.ShapeDtypeStruct((NP, D_IN), jnp.float32),
    ],
)


def _head_body(a0_ref, a1_ref, y0_ref, y1_ref, dinv_ref, b2_ref,
               wf1_ref, bf1_ref, wf2_ref, bf2_ref, o_ref):
    a0 = a0_ref[0] + a0_ref[1] + y0_ref[...]
    a1 = a1_ref[0] + a1_ref[1] + y1_ref[...]
    agg = jnp.concatenate([a0, a1], axis=1)
    h2 = jnp.maximum(agg * dinv_ref[...] + b2_ref[...], 0.0)
    h3 = jnp.dot(h2, wf1_ref[...], preferred_element_type=jnp.float32) + bf1_ref[...]
    h3 = jnp.maximum(h3, 0.0)
    o_ref[...] = jnp.dot(h3, wf2_ref[...], preferred_element_type=jnp.float32) + bf2_ref[...]


_head_call = pl.pallas_call(
    _head_body,
    grid=(NG,),
    in_specs=[
        pl.BlockSpec((NC, GB, D_IN), lambda i: (0, i, 0)),
        pl.BlockSpec((NC, GB, D_IN), lambda i: (0, i, 0)),
        pl.BlockSpec((GB, D_IN), lambda i: (i, 0)),
        pl.BlockSpec((GB, D_IN), lambda i: (i, 0)),
        pl.BlockSpec((GB, 1), lambda i: (i, 0)),
        pl.BlockSpec((1, D_HID), lambda i: (0, 0)),
        pl.BlockSpec((D_HID, D_HID // 2), lambda i: (0, 0)),
        pl.BlockSpec((1, D_HID // 2), lambda i: (0, 0)),
        pl.BlockSpec((D_HID // 2, 1), lambda i: (0, 0)),
        pl.BlockSpec((1, 1), lambda i: (0, 0)),
    ],
    out_specs=pl.BlockSpec((GB, 1), lambda i: (i, 0)),
    out_shape=jax.ShapeDtypeStruct((NP, 1), jnp.float32),
)


def kernel(x, edge_index, W1, b1, W2, b2, Wf1, bf1, Wf2, bf2):
    ei = edge_index.astype(jnp.int32)
    pad = N + (jnp.arange(EP - E, dtype=jnp.int32) % (NP - N))
    srcp = jnp.concatenate([ei[0], pad]).reshape(NW, NB, B)
    dstp = jnp.concatenate([ei[1], pad]).reshape(NW, NB, B)
    x_pad = jnp.zeros((NP, D_IN), jnp.float32).at[:N].set(x)

    xw1 = _mm1_call(x_pad, W1)
    degp = _deg_call(dstp)
    dinv, y10, y11 = _prep_call(degp, xw1)
    a10p, a11p = _seg_call(y10, y11, srcp, dstp)
    y20, y21 = _mid_call(a10p, a11p, y10, y11, dinv, b1.reshape(1, -1), W2)
    a20p, a21p = _seg_call(y20, y21, srcp, dstp)
    o = _head_call(a20p, a21p, y20, y21, dinv, b2.reshape(1, -1),
                   Wf1, bf1.reshape(1, -1), Wf2, bf2.reshape(1, -1))
    return o[:N, 0]


# speedup vs baseline: 1.2953x; 1.0013x over previous
import functools

import jax
import jax.numpy as jnp
from jax import lax
from jax.experimental import pallas as pl
from jax.experimental.pallas import tpu as pltpu
from jax.experimental.pallas import tpu_sc as plsc

N = 10000
NP = 10240
E = 320000
EP = 327680
NC = 2
NS = 16
NW = NC * NS
EPT = EP // NW
B = 128
NB = EPT // B
RPT = NP // NS
NBUF = 4
D_IN = 128
D_HID = 256
GB = 1024
NG = NP // GB

def _deg_body(dst_hbm, out_hbm, dst_v, buf_v, acc_sh, dsem):
    c = lax.axis_index("c")
    s = lax.axis_index("s")
    wid = s * NC + c

    fz = jnp.zeros((16,), jnp.float32)
    fo = jnp.ones((16,), jnp.float32)

    def zloop(i, _):
        buf_v[i // 8, pl.ds((i % 8) * 16, 16)] = fz
        return 0

    lax.fori_loop(0, B * D_IN // 16, zloop, 0)

    for j in range(RPT // B):
        pltpu.sync_copy(buf_v, acc_sh.at[pl.ds(s * RPT + j * B, B)])

    def oloop(i, _):
        buf_v[i // 8, pl.ds((i % 8) * 16, 16)] = fo
        return 0

    lax.fori_loop(0, B * D_IN // 16, oloop, 0)

    pltpu.sync_copy(dst_hbm.at[wid], dst_v)
    plsc.subcore_barrier()

    for j in range(NBUF):
        pltpu.async_copy(buf_v, acc_sh.at[dst_v.at[j]], dsem, add=True)

    def dloop(b, _):
        pltpu.make_async_copy(buf_v, acc_sh.at[dst_v.at[b]], dsem).wait()
        pltpu.async_copy(buf_v, acc_sh.at[dst_v.at[b + NBUF]], dsem, add=True)
        return 0

    lax.fori_loop(0, NB - NBUF, dloop, 0)
    for j in range(NBUF):
        pltpu.make_async_copy(buf_v, acc_sh.at[dst_v.at[j]], dsem).wait()
    plsc.subcore_barrier()

    pltpu.sync_copy(acc_sh.at[pl.ds(s * RPT, RPT)], out_hbm.at[c, pl.ds(s * RPT, RPT)])


CHK = 16
CHN = NB // CHK


def _seg_body(t0_hbm, t1_hbm, src_hbm, dst_hbm, o0_hbm, o1_hbm, src_v, dst_v,
              buf0, buf1, acc_sh, g0, g1, s0, s1):
    c = lax.axis_index("c")
    s = lax.axis_index("s")
    wid = s * NC + c

    fz = jnp.zeros((16,), jnp.float32)

    pltpu.sync_copy(src_hbm.at[wid], src_v)

    def do_pass(table_hbm, out_hbm):
        def g_issue(b, buf, sem):
            pltpu.async_copy(table_hbm.at[src_v.at[b]], buf, sem)

        def g_wait(b, buf, sem):
            pltpu.make_async_copy(table_hbm.at[src_v.at[b]], buf, sem).wait()

        def s_issue(q, kk, buf, sem):
            pltpu.async_copy(buf, acc_sh.at[dst_v.at[q, kk]], sem, add=True)

        def s_wait(q, kk, buf, sem):
            pltpu.make_async_copy(buf, acc_sh.at[dst_v.at[q, kk]], sem).wait()

        def zloop(i, _):
            buf0[i // 8, pl.ds((i % 8) * 16, 16)] = fz
            return 0

        lax.fori_loop(0, B * D_IN // 16, zloop, 0)

        for j in range(RPT // B):
            pltpu.sync_copy(buf0, acc_sh.at[pl.ds(s * RPT + j * B, B)])

        g_issue(0, buf0, g0)
        plsc.subcore_barrier()

        def outer(ch, _):
            q = ch % 2
            pltpu.sync_copy(dst_hbm.at[wid, pl.ds(ch * CHK, CHK)], dst_v.at[q])

            def inner(k, _):
                b0 = ch * CHK + 2 * k
                b1 = b0 + 1
                g_wait(b0, buf0, g0)
                s_issue(q, 2 * k, buf0, s0)

                @pl.when(b0 > 0)
                def _():
                    s_wait(q, 2 * k, buf1, s1)

                g_issue(b1, buf1, g1)
                g_wait(b1, buf1, g1)
                s_issue(q, 2 * k + 1, buf1, s1)
                s_wait(q, 2 * k, buf0, s0)

                @pl.when(b1 + 1 < NB)
                def _():
                    g_issue(b1 + 1, buf0, g0)

                return 0

            lax.fori_loop(0, CHK // 2, inner, 0)
            return 0

        lax.fori_loop(0, CHN, outer, 0)
        s_wait(0, 0, buf1, s1)
        plsc.subcore_barrier()

        pltpu.sync_copy(acc_sh.at[pl.ds(s * RPT, RPT)],
                        out_hbm.at[c, pl.ds(s * RPT, RPT)])

    do_pass(t0_hbm, o0_hbm)
    do_pass(t1_hbm, o1_hbm)


@functools.cache
def _sc_calls():
    mesh = plsc.VectorSubcoreMesh(
        core_axis_name="c", subcore_axis_name="s", num_cores=NC, num_subcores=NS
    )
    deg_call = pl.kernel(
        _deg_body,
        out_type=jax.ShapeDtypeStruct((NC, NP, D_IN), jnp.float32),
        mesh=mesh,
        scratch_types=[
            pltpu.VMEM((NB, B), jnp.int32),
            pltpu.VMEM((B, D_IN), jnp.float32),
            pltpu.VMEM_SHARED((NP, D_IN), jnp.float32),
            pltpu.SemaphoreType.DMA,
        ],
    )
    seg_call = pl.kernel(
        _seg_body,
        out_type=(
            jax.ShapeDtypeStruct((NC, NP, D_IN), jnp.float32),
            jax.ShapeDtypeStruct((NC, NP, D_IN), jnp.float32),
        ),
        mesh=mesh,
        scratch_types=[
            pltpu.VMEM((NB, B), jnp.int32),
            pltpu.VMEM((2, CHK, B), jnp.int32),
            pltpu.VMEM((B, D_IN), jnp.float32),
            pltpu.VMEM((B, D_IN), jnp.float32),
            pltpu.VMEM_SHARED((NP, D_IN), jnp.float32),
            pltpu.SemaphoreType.DMA,
            pltpu.SemaphoreType.DMA,
            pltpu.SemaphoreType.DMA,
            pltpu.SemaphoreType.DMA,
        ],
    )
    return deg_call, seg_call


def _mm1_body(x_ref, w1_ref, xw_ref):
    xw_ref[...] = jnp.dot(x_ref[...], w1_ref[...],
                          preferred_element_type=jnp.float32)


_mm1_call = pl.pallas_call(
    _mm1_body,
    grid=(NG,),
    in_specs=[
        pl.BlockSpec((GB, D_IN), lambda i: (i, 0)),
        pl.BlockSpec((D_IN, D_HID), lambda i: (0, 0)),
    ],
    out_specs=pl.BlockSpec((GB, D_HID), lambda i: (i, 0)),
    out_shape=jax.ShapeDtypeStruct((NP, D_HID), jnp.float32),
)


def _prep_body(dp_ref, xw_ref, dinv_ref, y0_ref, y1_ref):
    deg = dp_ref[0][:, :1] + dp_ref[1][:, :1] + 1.0
    dinv = 1.0 / jnp.sqrt(deg)
    dinv_ref[...] = dinv
    y = xw_ref[...] * dinv
    y0_ref[...] = y[:, :D_IN]
    y1_ref[...] = y[:, D_IN:]


_prep_call = pl.pallas_call(
    _prep_body,
    grid=(NG,),
    in_specs=[
        pl.BlockSpec((NC, GB, D_IN), lambda i: (0, i, 0)),
        pl.BlockSpec((GB, D_HID), lambda i: (i, 0)),
    ],
    out_specs=[
        pl.BlockSpec((GB, 1), lambda i: (i, 0)),
        pl.BlockSpec((GB, D_IN), lambda i: (i, 0)),
        pl.BlockSpec((GB, D_IN), lambda i: (i, 0)),
    ],
    out_shape=[
        jax.ShapeDtypeStruct((NP, 1), jnp.float32),
        jax.ShapeDtypeStruct((NP, D_IN), jnp.float32),
        jax.ShapeDtypeStruct((NP, D_IN), jnp.float32),
    ],
)


def _mid_body(a0_ref, a1_ref, y0_ref, y1_ref, dinv_ref, b1_ref, w2_ref,
              t0_ref, t1_ref):
    a0 = a0_ref[0] + a0_ref[1] + y0_ref[...]
    a1 = a1_ref[0] + a1_ref[1] + y1_ref[...]
    agg = jnp.concatenate([a0, a1], axis=1)
    h1 = jnp.maximum(agg * dinv_ref[...] + b1_ref[...], 0.0)
    xw = jnp.dot(h1, w2_ref[...], preferred_element_type=jnp.float32)
    t = xw * dinv_ref[...]
    t0_ref[...] = t[:, :D_IN]
    t1_ref[...] = t[:, D_IN:]


_mid_call = pl.pallas_call(
    _mid_body,
    grid=(NG,),
    in_specs=[
        pl.BlockSpec((NC, GB, D_IN), lambda i: (0, i, 0)),
        pl.BlockSpec((NC, GB, D_IN), lambda i: (0, i, 0)),
        pl.BlockSpec((GB, D_IN), lambda i: (i, 0)),
        pl.BlockSpec((GB, D_IN), lambda i: (i, 0)),
        pl.BlockSpec((GB, 1), lambda i: (i, 0)),
        pl.BlockSpec((1, D_HID), lambda i: (0, 0)),
        pl.BlockSpec((D_HID, D_HID), lambda i: (0, 0)),
    ],
    out_specs=[
        pl.BlockSpec((GB, D_IN), lambda i: (i, 0)),
        pl.BlockSpec((GB, D_IN), lambda i: (i, 0)),
    ],
    out_shape=[
        jax.ShapeDtypeStruct((NP, D_IN), jnp.float32),
        jax.ShapeDtypeStruct((NP, D_IN), jnp.float32),
    ],
)


def _head_body(a0_ref, a1_ref, y0_ref, y1_ref, dinv_ref, b2_ref,
               wf1_ref, bf1_ref, wf2_ref, bf2_ref, o_ref):
    a0 = a0_ref[0] + a0_ref[1] + y0_ref[...]
    a1 = a1_ref[0] + a1_ref[1] + y1_ref[...]
    agg = jnp.concatenate([a0, a1], axis=1)
    h2 = jnp.maximum(agg * dinv_ref[...] + b2_ref[...], 0.0)
    h3 = jnp.dot(h2, wf1_ref[...], preferred_element_type=jnp.float32) + bf1_ref[...]
    h3 = jnp.maximum(h3, 0.0)
    o_ref[...] = jnp.dot(h3, wf2_ref[...], preferred_element_type=jnp.float32) + bf2_ref[...]


_head_call = pl.pallas_call(
    _head_body,
    grid=(NG,),
    in_specs=[
        pl.BlockSpec((NC, GB, D_IN), lambda i: (0, i, 0)),
        pl.BlockSpec((NC, GB, D_IN), lambda i: (0, i, 0)),
        pl.BlockSpec((GB, D_IN), lambda i: (i, 0)),
        pl.BlockSpec((GB, D_IN), lambda i: (i, 0)),
        pl.BlockSpec((GB, 1), lambda i: (i, 0)),
        pl.BlockSpec((1, D_HID), lambda i: (0, 0)),
        pl.BlockSpec((D_HID, D_HID // 2), lambda i: (0, 0)),
        pl.BlockSpec((1, D_HID // 2), lambda i: (0, 0)),
        pl.BlockSpec((D_HID // 2, 1), lambda i: (0, 0)),
        pl.BlockSpec((1, 1), lambda i: (0, 0)),
    ],
    out_specs=pl.BlockSpec((GB, 1), lambda i: (i, 0)),
    out_shape=jax.ShapeDtypeStruct((NP, 1), jnp.float32),
)


def kernel(x, edge_index, W1, b1, W2, b2, Wf1, bf1, Wf2, bf2):
    ei = edge_index.astype(jnp.int32)
    pad = N + (jnp.arange(EP - E, dtype=jnp.int32) % (NP - N))
    srcp = jnp.concatenate([ei[0], pad]).reshape(NW, NB, B)
    dstp = jnp.concatenate([ei[1], pad]).reshape(NW, NB, B)
    x_pad = jnp.zeros((NP, D_IN), jnp.float32).at[:N].set(x)

    _deg_call, _seg_call = _sc_calls()
    xw1 = _mm1_call(x_pad, W1)
    degp = _deg_call(dstp)
    dinv, y10, y11 = _prep_call(degp, xw1)
    a10p, a11p = _seg_call(y10, y11, srcp, dstp)
    y20, y21 = _mid_call(a10p, a11p, y10, y11, dinv, b1.reshape(1, -1), W2)
    a20p, a21p = _seg_call(y20, y21, srcp, dstp)
    o = _head_call(a20p, a21p, y20, y21, dinv, b2.reshape(1, -1),
                   Wf1, bf1.reshape(1, -1), Wf2, bf2.reshape(1, -1))
    return o[:N, 0]
